# Initial kernel scaffold; baseline (speedup 1.0000x reference)
import jax
import jax.numpy as jnp
from jax.experimental import pallas as pl


def kernel(x, edge_index, W1, b1, W2, b2):
    raise NotImplementedError("write your pallas kernel here")


# speedup vs baseline: 23.2469x; 23.2469x over previous
import functools

import jax
import jax.numpy as jnp
from jax import lax
from jax.experimental import pallas as pl
from jax.experimental.pallas import tpu as pltpu
from jax.experimental.pallas import tpu_sc as plsc

N = 10000
E = 320000
F_IN = 128
H = 128
OUT = 64

NPAD = 10240
NC = 2
NS = 16
NW = NC * NS
EPT = E // NW
CHUNK = 125
NCH = EPT // CHUNK
RPT = NPAD // NS
ZR = 32

_mesh = plsc.VectorSubcoreMesh(core_axis_name="c", subcore_axis_name="s")
_sc_params = pltpu.CompilerParams(needs_layout_passes=False)


def _deg_body(col_hbm, out_hbm, colbuf, hist):
    cid = lax.axis_index("c")
    sid = lax.axis_index("s")
    wid = sid * NC + cid
    pltpu.sync_copy(col_hbm.at[wid], colbuf)

    def zero(j, carry):
        hist[pl.ds(j * 16, 16)] = jnp.zeros((16,), jnp.float32)
        return carry

    lax.fori_loop(0, NPAD // 16, zero, 0)

    ones = jnp.ones((16,), jnp.float32)

    def body(j, carry):
        idx = colbuf[pl.ds(j * 16, 16)]
        plsc.addupdate_scatter(hist, [idx], ones)
        return carry

    lax.fori_loop(0, EPT // 16, body, 0)
    pltpu.sync_copy(hist, out_hbm.at[wid])


_deg = pl.kernel(
    _deg_body,
    mesh=_mesh,
    out_type=jax.ShapeDtypeStruct((NW, NPAD), jnp.float32),
    scratch_types=[
        pltpu.VMEM((EPT,), jnp.int32),
        pltpu.VMEM((NPAD,), jnp.float32),
    ],
    compiler_params=_sc_params,
)


def _scatter_body(feat, g_hbm, row_hbm, col_hbm, out_hbm,
                  rowbuf, colbuf, gbuf, zbuf, s_sh, gsem):
    cid = lax.axis_index("c")
    sid = lax.axis_index("s")
    wid = sid * NC + cid
    pltpu.sync_copy(row_hbm.at[wid], rowbuf)
    pltpu.sync_copy(col_hbm.at[wid], colbuf)

    def zrow(r, carry):
        def zcol(c, carry2):
            zbuf[r, pl.ds(c * 16, 16)] = jnp.zeros((16,), jnp.float32)
            return carry2

        return lax.fori_loop(0, feat // 16, zcol, carry)

    lax.fori_loop(0, ZR, zrow, 0)

    def zfill(r, carry):
        pltpu.sync_copy(zbuf, s_sh.at[pl.ds(sid * RPT + r * ZR, ZR)])
        return carry

    lax.fori_loop(0, RPT // ZR, zfill, 0)
    plsc.subcore_barrier()

    def body(j, carry):
        pltpu.async_copy(g_hbm.at[rowbuf.at[j]], gbuf, gsem).wait()
        pltpu.sync_copy(gbuf, s_sh.at[colbuf.at[j]], add=True)
        return carry

    lax.fori_loop(0, NCH, body, 0)
    plsc.subcore_barrier()

    def out(r, carry):
        sl = pl.ds(sid * RPT + r * ZR, ZR)
        pltpu.sync_copy(s_sh.at[sl], out_hbm.at[cid, sl])
        return carry

    lax.fori_loop(0, RPT // ZR, out, 0)


def _make_scatter(feat):
    return pl.kernel(
        functools.partial(_scatter_body, feat),
        mesh=_mesh,
        out_type=jax.ShapeDtypeStruct((NC, NPAD, feat), jnp.float32),
        scratch_types=[
            pltpu.VMEM((NCH, CHUNK), jnp.int32),
            pltpu.VMEM((NCH, CHUNK), jnp.int32),
            pltpu.VMEM((CHUNK, feat), jnp.float32),
            pltpu.VMEM((ZR, feat), jnp.float32),
            pltpu.VMEM_SHARED((NPAD, feat), jnp.float32),
            pltpu.SemaphoreType.DMA,
        ],
        compiler_params=_sc_params,
    )


_scatter_h = _make_scatter(H)


NB = 1024


def _dinv_of(hist_ref):
    deg = jnp.sum(hist_ref[...], axis=0) + 1.0
    return lax.rsqrt(deg)[:, None]


def _tc1_body(hist_ref, x_ref, w_ref, g_ref):
    h = jnp.dot(x_ref[...], w_ref[...], preferred_element_type=jnp.float32)
    g_ref[...] = h * _dinv_of(hist_ref)


def _tc2_body(hist_ref, s_ref, g1_ref, b1_ref, w2_ref, g2_ref):
    dinv = _dinv_of(hist_ref)
    s = s_ref[0] + s_ref[1] + g1_ref[...]
    h1 = jnp.maximum(s * dinv + b1_ref[...], 0.0)
    g2_ref[...] = jnp.dot(h1, w2_ref[...],
                          preferred_element_type=jnp.float32) * dinv


def _tc3_body(hist_ref, s_ref, g2_ref, b2_ref, z_ref):
    s = s_ref[0] + s_ref[1] + g2_ref[...]
    z_ref[...] = s * _dinv_of(hist_ref) + b2_ref[...]


_GRID = (NPAD // NB,)
_hist_spec = pl.BlockSpec((NW, NB), lambda i: (0, i))


def _row_spec(f):
    return pl.BlockSpec((NB, f), lambda i: (i, 0))


def _part_spec(f):
    return pl.BlockSpec((NC, NB, f), lambda i: (0, i, 0))


def _full_spec(r, c):
    return pl.BlockSpec((r, c), lambda i: (0, 0))


_tc1 = pl.pallas_call(
    _tc1_body,
    grid=_GRID,
    in_specs=[_hist_spec, _row_spec(F_IN), _full_spec(F_IN, H)],
    out_specs=_row_spec(H),
    out_shape=jax.ShapeDtypeStruct((NPAD, H), jnp.float32),
)

_tc2 = pl.pallas_call(
    _tc2_body,
    grid=_GRID,
    in_specs=[_hist_spec, _part_spec(H), _row_spec(H),
              _full_spec(1, H), _full_spec(H, H)],
    out_specs=_row_spec(H),
    out_shape=jax.ShapeDtypeStruct((NPAD, H), jnp.float32),
)

_tc3 = pl.pallas_call(
    _tc3_body,
    grid=_GRID,
    in_specs=[_hist_spec, _part_spec(H), _row_spec(H), _full_spec(1, H)],
    out_specs=_row_spec(H),
    out_shape=jax.ShapeDtypeStruct((NPAD, H), jnp.float32),
)


def kernel(x, edge_index, W1, b1, W2, b2):
    ei = edge_index.astype(jnp.int32)
    row = ei[0]
    col = ei[1]
    col_t = col.reshape(NW, EPT)
    row3 = row.reshape(NW, NCH, CHUNK)
    col3 = col.reshape(NW, NCH, CHUNK)
    xp = jnp.pad(x, ((0, NPAD - N), (0, 0)))

    w2p = jnp.pad(W2, ((0, 0), (0, H - OUT)))
    b2p = jnp.pad(b2, (0, H - OUT)).reshape(1, H)

    hist = _deg(col_t)
    g1 = _tc1(hist, xp, W1)
    s1 = _scatter_h(g1, row3, col3)
    g2 = _tc2(hist, s1, g1, b1.reshape(1, H), w2p)
    s2 = _scatter_h(g2, row3, col3)
    z = _tc3(hist, s2, g2, b2p)
    return z[:N, :OUT]
